# Initial kernel scaffold; baseline (speedup 1.0000x reference)
import jax
import jax.numpy as jnp
from jax.experimental import pallas as pl


def kernel(z, embedding):
    raise NotImplementedError("write your pallas kernel here")


# speedup vs baseline: 1.7454x; 1.7454x over previous
import functools

import jax
import jax.numpy as jnp
from jax.experimental import pallas as pl
from jax.experimental.pallas import tpu as pltpu

_K = 1024
_D = 64
_BETA = 0.25
_BS = 512


def _vq_body(z_ref, e_ref, zq_ref, idx_ref, loss_ref, perp_ref, counts_ref):
    b = pl.program_id(0)
    j = pl.program_id(1)
    nb = pl.num_programs(0)
    nj = pl.num_programs(1)

    zb = z_ref[0]
    e = e_ref[...]

    esq = jnp.sum(e * e, axis=1, keepdims=True)
    zsq = jnp.sum(zb * zb, axis=0, keepdims=True)
    mm = jax.lax.dot_general(
        e, zb, (((1,), (0,)), ((), ())),
        preferred_element_type=jnp.float32)
    dist = (zsq + esq) - 2.0 * mm

    kiota = jax.lax.broadcasted_iota(jnp.int32, dist.shape, 0)
    minv = jnp.min(dist, axis=0, keepdims=True)
    cand = jnp.where(dist == minv, kiota, _K)
    idx = jnp.min(cand, axis=0).astype(jnp.int32)
    idx_ref[0, 0, :] = idx

    oh = (kiota == idx[None, :]).astype(jnp.float32)
    zq = jax.lax.dot_general(
        e, oh, (((0,), (0,)), ((), ())),
        preferred_element_type=jnp.float32)
    zq_ref[0] = zb + (zq - zb)

    diff = zb - zq
    part_loss = jnp.sum(diff * diff).reshape(1, 1)
    part_counts = jnp.sum(oh, axis=1, keepdims=True)

    first = jnp.logical_and(b == 0, j == 0)

    @pl.when(first)
    def _init():
        counts_ref[...] = part_counts
        loss_ref[...] = part_loss

    @pl.when(jnp.logical_not(first))
    def _acc():
        counts_ref[...] = counts_ref[...] + part_counts
        loss_ref[...] = loss_ref[...] + part_loss

    @pl.when(jnp.logical_and(b == nb - 1, j == nj - 1))
    def _fin():
        n_total = nb * nj * _BS
        avg = counts_ref[...] * (1.0 / n_total)
        ent = jnp.sum(avg * jnp.log(avg + 1e-10)).reshape(1, 1)
        perp_ref[...] = jnp.exp(-ent)
        loss_ref[...] = loss_ref[...] * (_BETA / (n_total * _D))


@functools.partial(jax.jit, static_argnames=("interpret",))
def _vq(z, embedding, interpret=False):
    bsz, c, dd, h, w = z.shape
    s = dd * h * w
    zr = z.reshape(bsz, c, s)
    nj = s // _BS
    grid = (bsz, nj)

    zq, idxb, loss, perp = pl.pallas_call(
        _vq_body,
        grid=grid,
        in_specs=[
            pl.BlockSpec((1, c, _BS), lambda b, j: (b, 0, j)),
            pl.BlockSpec((_K, _D), lambda b, j: (0, 0)),
        ],
        out_specs=[
            pl.BlockSpec((1, c, _BS), lambda b, j: (b, 0, j)),
            pl.BlockSpec((1, 1, _BS), lambda b, j: (b * nj + j, 0, 0)),
            pl.BlockSpec((1, 1), lambda b, j: (0, 0)),
            pl.BlockSpec((1, 1), lambda b, j: (0, 0)),
        ],
        out_shape=[
            jax.ShapeDtypeStruct((bsz, c, s), jnp.float32),
            jax.ShapeDtypeStruct((bsz * nj, 1, _BS), jnp.int32),
            jax.ShapeDtypeStruct((1, 1), jnp.float32),
            jax.ShapeDtypeStruct((1, 1), jnp.float32),
        ],
        scratch_shapes=[pltpu.VMEM((_K, 1), jnp.float32)],
        interpret=interpret,
    )(zr, embedding)

    z_q_out = zq.reshape(bsz, c, dd, h, w)
    indices = idxb.reshape(bsz, dd, h, w)
    return z_q_out, loss[0, 0], indices, perp[0, 0]


def kernel(z, embedding):
    return _vq(z, embedding)
